# Initial kernel scaffold; baseline (speedup 1.0000x reference)
import jax
import jax.numpy as jnp
from jax.experimental import pallas as pl


def kernel(hidden_states, gate_w, w1, w2, w3, num_global_tokens, max_num_tokens_per_gpu):
    raise NotImplementedError("write your pallas kernel here")


# speedup vs baseline: 1.6534x; 1.6534x over previous
import jax
import jax.numpy as jnp
from jax.experimental import pallas as pl
from jax.experimental.pallas import tpu as pltpu


def _route_body(x_ref, gw_ref, r_ref):
    x = x_ref[...]
    logits = jax.lax.dot_general(
        x, gw_ref[...], (((1,), (1,)), ((), ())),
        preferred_element_type=jnp.float32)
    E = logits.shape[-1]
    m = jnp.max(logits, axis=-1, keepdims=True)
    p = jnp.exp(logits - m)
    iota = jax.lax.broadcasted_iota(jnp.int32, p.shape, 1)
    m1 = jnp.max(p, axis=-1, keepdims=True)
    i1 = jnp.min(jnp.where(p == m1, iota, E), axis=-1, keepdims=True)
    mask1 = iota == i1
    p2 = jnp.where(mask1, -jnp.inf, p)
    m2 = jnp.max(p2, axis=-1, keepdims=True)
    i2 = jnp.min(jnp.where(p2 == m2, iota, E), axis=-1, keepdims=True)
    mask2 = iota == i2
    s = m1 + m2
    r_ref[...] = jnp.where(mask1, m1 / s, 0.0) + jnp.where(mask2, m2 / s, 0.0)


def _moe_body(x_ref, r_ref, w1_ref, w3_ref, w2_ref, o_ref):
    e = pl.program_id(1)
    x = x_ref[...]
    w1 = w1_ref[0]
    w3 = w3_ref[0]
    w2 = w2_ref[0]
    g = jax.lax.dot_general(x, w1, (((1,), (1,)), ((), ())),
                            preferred_element_type=jnp.float32)
    u = jax.lax.dot_general(x, w3, (((1,), (1,)), ((), ())),
                            preferred_element_type=jnp.float32)
    h = g * jax.lax.logistic(g) * u
    y = jax.lax.dot_general(h, w2, (((1,), (1,)), ((), ())),
                            preferred_element_type=jnp.float32)
    r = r_ref[...]
    iota = jax.lax.broadcasted_iota(jnp.int32, r.shape, 1)
    rw = jnp.sum(jnp.where(iota == e, r, 0.0), axis=1, keepdims=True)
    contrib = rw * y

    @pl.when(e == 0)
    def _():
        o_ref[...] = contrib

    @pl.when(e != 0)
    def _():
        o_ref[...] += contrib


def kernel(hidden_states, gate_w, w1, w2, w3, num_global_tokens,
           max_num_tokens_per_gpu):
    T, D = hidden_states.shape
    E, FF, _ = w1.shape
    x = hidden_states.astype(jnp.float32)

    BT_R = 256
    routing = pl.pallas_call(
        _route_body,
        grid=(T // BT_R,),
        in_specs=[
            pl.BlockSpec((BT_R, D), lambda t: (t, 0)),
            pl.BlockSpec((E, D), lambda t: (0, 0)),
        ],
        out_specs=pl.BlockSpec((BT_R, E), lambda t: (t, 0)),
        out_shape=jax.ShapeDtypeStruct((T, E), jnp.float32),
    )(x, gate_w)

    BT = 512
    out = pl.pallas_call(
        _moe_body,
        grid=(T // BT, E),
        in_specs=[
            pl.BlockSpec((BT, D), lambda t, e: (t, 0)),
            pl.BlockSpec((BT, E), lambda t, e: (t, 0)),
            pl.BlockSpec((1, FF, D), lambda t, e: (e, 0, 0)),
            pl.BlockSpec((1, FF, D), lambda t, e: (e, 0, 0)),
            pl.BlockSpec((1, D, FF), lambda t, e: (e, 0, 0)),
        ],
        out_specs=pl.BlockSpec((BT, D), lambda t, e: (t, 0)),
        out_shape=jax.ShapeDtypeStruct((T, D), jnp.float32),
        compiler_params=pltpu.CompilerParams(
            dimension_semantics=("parallel", "arbitrary")),
    )(x, routing, w1, w3, w2)
    return out
